# Initial kernel scaffold; baseline (speedup 1.0000x reference)
import jax
import jax.numpy as jnp
from jax.experimental import pallas as pl


def kernel(x, edge_index, edge_attr, batch, params):
    raise NotImplementedError("write your pallas kernel here")


# speedup vs baseline: 1.0073x; 1.0073x over previous
import jax
import jax.numpy as jnp
from jax.experimental import pallas as pl

N = 50000
E = 800000
D_IN = 128
H = 64
OUT = 64
NUM_GRAPHS = 64
BN_EPS = 1e-5


def _head_pallas(g_sums, cnt, lin1_w, lin1_b, lin2_w, lin2_b):
    def body(s_ref, c_ref, w1_ref, b1_ref, w2_ref, b2_ref, o_ref):
        g = s_ref[...] / jnp.maximum(c_ref[...], 1.0)
        g = jnp.maximum(jnp.dot(g, w1_ref[...]) + b1_ref[...], 0.0)
        o_ref[...] = jnp.dot(g, w2_ref[...]) + b2_ref[...]

    return pl.pallas_call(
        body,
        out_shape=jax.ShapeDtypeStruct((NUM_GRAPHS, OUT), jnp.float32),
    )(g_sums, cnt, lin1_w, lin1_b, lin2_w, lin2_b)


def kernel(x, edge_index, edge_attr, batch, params):
    h = x
    src = edge_index[0]
    dst = edge_index[1]
    for p in params["layers"]:
        e = edge_attr @ p["we"] + p["be"]
        m = jax.nn.relu(h[src] + e)
        agg = jnp.zeros_like(h).at[dst].add(m)
        z = h + agg
        z = jax.nn.relu(z @ p["w1"] + p["b1"])
        z = jax.nn.relu(z @ p["w2"] + p["b2"])
        h = z / jnp.sqrt(1.0 + BN_EPS) * p["gamma"] + p["beta"]
    sums = jax.ops.segment_sum(h, batch, num_segments=NUM_GRAPHS)
    cnt = jax.ops.segment_sum(jnp.ones((h.shape[0],), jnp.float32), batch,
                              num_segments=NUM_GRAPHS)
    return _head_pallas(sums, cnt[:, None], params["lin1_w"], params["lin1_b"],
                        params["lin2_w"], params["lin2_b"])


# speedup vs baseline: 1.2770x; 1.2678x over previous
import functools

import jax
import jax.numpy as jnp
from jax import lax
from jax.experimental import pallas as pl
from jax.experimental.pallas import tpu as pltpu
from jax.experimental.pallas import tpu_sc as plsc

N = 50000
E = 800000
D_IN = 128
D_E = 16
H = 64
OUT = 64
NUM_GRAPHS = 64
BN_EPS = 1e-5

FC = 32
W = 80
NUM_SC = 2
NUM_TILES = 16
EPT = E // NUM_TILES
NW = EPT // W


def _chunk_body(nc, x_ref, o_ref):
    for c in range(nc):
        o_ref[c] = x_ref[:, c * FC:(c + 1) * FC]


def _to_chunked(x):
    n, d = x.shape
    nc = d // FC
    bn = 2000
    return pl.pallas_call(
        functools.partial(_chunk_body, nc),
        grid=(n // bn,),
        in_specs=[pl.BlockSpec((bn, d), lambda i: (i, 0))],
        out_specs=pl.BlockSpec((nc, bn, FC), lambda i: (0, i, 0)),
        out_shape=jax.ShapeDtypeStruct((nc, n, FC), jnp.float32),
    )(x)


def _edge_e_body(nc, a_ref, w_ref, b_ref, o_ref):
    y = jax.lax.dot(a_ref[...], w_ref[...],
                    precision=jax.lax.Precision.HIGHEST) + b_ref[...]
    for c in range(nc):
        o_ref[c] = y[:, c * FC:(c + 1) * FC]


def _edge_e(edge_attr, we, be):
    dout = we.shape[1]
    nc = dout // FC
    be_blk = 8000
    return pl.pallas_call(
        functools.partial(_edge_e_body, nc),
        grid=(E // be_blk,),
        in_specs=[
            pl.BlockSpec((be_blk, D_E), lambda i: (i, 0)),
            pl.BlockSpec((D_E, dout), lambda i: (0, 0)),
            pl.BlockSpec((1, dout), lambda i: (0, 0)),
        ],
        out_specs=pl.BlockSpec((nc, be_blk, FC), lambda i: (0, i, 0)),
        out_shape=jax.ShapeDtypeStruct((nc, E, FC), jnp.float32),
    )(edge_attr, we, be.reshape(1, dout))


def _sc_gather_scatter(h_chunked, e_chunked, src, dst, zeros_chunk):
    nc = h_chunked.shape[0]
    per_core = nc // NUM_SC
    mesh = plsc.VectorSubcoreMesh(core_axis_name="c", subcore_axis_name="s")

    @functools.partial(
        pl.kernel,
        out_type=jax.ShapeDtypeStruct((nc, N, FC), jnp.float32),
        mesh=mesh,
        scratch_types=[
            pltpu.VMEM_SHARED((N, FC), jnp.float32),
            pltpu.VMEM((W,), jnp.int32),
            pltpu.VMEM((W,), jnp.int32),
            pltpu.VMEM((W, FC), jnp.float32),
            pltpu.VMEM((W, FC), jnp.float32),
            pltpu.SemaphoreType.DMA,
        ],
        compiler_params=pltpu.CompilerParams(use_tc_tiling_on_sc=False),
    )
    def body(h_hbm, e_hbm, src_hbm, dst_hbm, z_hbm, agg_hbm,
             acc_sh, sidx, didx, hs, ev, sem):
        core = lax.axis_index("c")
        tid = lax.axis_index("s")
        tile_lo = tid * EPT

        for k in range(per_core):
            ci = core * per_core + k

            @pl.when(tid == 0)
            def _zero():
                pltpu.sync_copy(z_hbm, acc_sh)

            plsc.subcore_barrier()

            @pl.loop(0, NW)
            def _win(w):
                e0 = tile_lo + w * W
                pltpu.sync_copy(src_hbm.at[pl.ds(e0, W)], sidx)
                pltpu.sync_copy(dst_hbm.at[pl.ds(e0, W)], didx)
                gcp = pltpu.async_copy(h_hbm.at[ci].at[sidx], hs, sem)
                pltpu.sync_copy(e_hbm.at[ci].at[pl.ds(e0, W), :], ev)
                gcp.wait()

                @pl.loop(0, W)
                def _row(i):
                    for j in range(FC // 16):
                        sl = pl.ds(j * 16, 16)
                        hs[i, sl] = jnp.maximum(hs[i, sl] + ev[i, sl], 0.0)

                pltpu.sync_copy(hs, acc_sh.at[didx], add=True)

            plsc.subcore_barrier()

            @pl.when(tid == 0)
            def _flush():
                pltpu.sync_copy(acc_sh, agg_hbm.at[ci])

            plsc.subcore_barrier()

    return body(h_chunked, e_chunked, src, dst, zeros_chunk)


def _mlp_body(nc, h_ref, a_ref, w1_ref, b1_ref, w2_ref, b2_ref, g_ref, be_ref,
              o_ref):
    acc = jnp.zeros((h_ref.shape[1], H), jnp.float32)
    for c in range(nc):
        z = h_ref[c] + a_ref[c]
        acc = acc + jax.lax.dot(z, w1_ref[0, c],
                                precision=jax.lax.Precision.HIGHEST)
    y = jnp.maximum(acc + b1_ref[...], 0.0)
    y = jnp.maximum(jax.lax.dot(y, w2_ref[...],
                                precision=jax.lax.Precision.HIGHEST)
                    + b2_ref[...], 0.0)
    y = y * (1.0 / jnp.sqrt(1.0 + BN_EPS)) * g_ref[...] + be_ref[...]
    for c in range(H // FC):
        o_ref[c] = y[:, c * FC:(c + 1) * FC]


def _node_mlp(h_chunked, agg_chunked, p):
    nc = h_chunked.shape[0]
    din = nc * FC
    bn = 2000
    w1 = p["w1"].reshape(1, nc, FC, H)
    return pl.pallas_call(
        functools.partial(_mlp_body, nc),
        grid=(N // bn,),
        in_specs=[
            pl.BlockSpec((nc, bn, FC), lambda i: (0, i, 0)),
            pl.BlockSpec((nc, bn, FC), lambda i: (0, i, 0)),
            pl.BlockSpec((1, nc, FC, H), lambda i: (0, 0, 0, 0)),
            pl.BlockSpec((1, H), lambda i: (0, 0)),
            pl.BlockSpec((H, H), lambda i: (0, 0)),
            pl.BlockSpec((1, H), lambda i: (0, 0)),
            pl.BlockSpec((1, H), lambda i: (0, 0)),
            pl.BlockSpec((1, H), lambda i: (0, 0)),
        ],
        out_specs=pl.BlockSpec((H // FC, bn, FC), lambda i: (0, i, 0)),
        out_shape=jax.ShapeDtypeStruct((H // FC, N, FC), jnp.float32),
    )(h_chunked, agg_chunked, w1, p["b1"].reshape(1, H), p["w2"],
      p["b2"].reshape(1, H), p["gamma"].reshape(1, H), p["beta"].reshape(1, H))


def _pool_body(h_ref, b_ref, w1_ref, b1_ref, w2_ref, b2_ref, o_ref,
               sums_ref, cnt_ref):
    i = pl.program_id(0)

    @pl.when(i == 0)
    def _init():
        sums_ref[...] = jnp.zeros_like(sums_ref)
        cnt_ref[...] = jnp.zeros_like(cnt_ref)

    ids = b_ref[0]
    onehot = (ids == lax.broadcasted_iota(jnp.int32, (1, NUM_GRAPHS),
                                          1)).astype(jnp.float32)
    nc = h_ref.shape[0]
    for c in range(nc):
        sums_ref[:, c * FC:(c + 1) * FC] += jax.lax.dot_general(
            onehot, h_ref[c], (((0,), (0,)), ((), ())),
            precision=jax.lax.Precision.HIGHEST)
    ones_col = jnp.ones((onehot.shape[0], 8), jnp.float32)
    cnt_ref[...] += jax.lax.dot_general(
        onehot, ones_col, (((0,), (0,)), ((), ())),
        precision=jax.lax.Precision.HIGHEST)

    @pl.when(i == pl.num_programs(0) - 1)
    def _head():
        g = sums_ref[...] / jnp.maximum(cnt_ref[:, 0:1], 1.0)
        g = jnp.maximum(jax.lax.dot(g, w1_ref[...],
                                    precision=jax.lax.Precision.HIGHEST)
                        + b1_ref[...], 0.0)
        o_ref[...] = jax.lax.dot(g, w2_ref[...],
                                 precision=jax.lax.Precision.HIGHEST) \
            + b2_ref[...]


def _pool_head(h_chunked, batch, params):
    nc = h_chunked.shape[0]
    bn = 2000
    nb = N // bn
    batch3 = batch.reshape(nb, bn, 1)
    return pl.pallas_call(
        _pool_body,
        grid=(nb,),
        in_specs=[
            pl.BlockSpec((nc, bn, FC), lambda i: (0, i, 0)),
            pl.BlockSpec((1, bn, 1), lambda i: (i, 0, 0)),
            pl.BlockSpec((H, H), lambda i: (0, 0)),
            pl.BlockSpec((1, H), lambda i: (0, 0)),
            pl.BlockSpec((H, OUT), lambda i: (0, 0)),
            pl.BlockSpec((1, OUT), lambda i: (0, 0)),
        ],
        out_specs=pl.BlockSpec((NUM_GRAPHS, OUT), lambda i: (0, 0)),
        out_shape=jax.ShapeDtypeStruct((NUM_GRAPHS, OUT), jnp.float32),
        scratch_shapes=[
            pltpu.VMEM((NUM_GRAPHS, H), jnp.float32),
            pltpu.VMEM((NUM_GRAPHS, 8), jnp.float32),
        ],
    )(h_chunked, batch3, params["lin1_w"], params["lin1_b"].reshape(1, H),
      params["lin2_w"], params["lin2_b"].reshape(1, OUT))


def kernel(x, edge_index, edge_attr, batch, params):
    src = edge_index[0]
    dst = edge_index[1]
    zeros_chunk = jnp.zeros((N, FC), jnp.float32)

    h = _to_chunked(x)
    for p in params["layers"]:
        e = _edge_e(edge_attr, p["we"], p["be"])
        agg = _sc_gather_scatter(h, e, src, dst, zeros_chunk)
        h = _node_mlp(h, agg, p)
    return _pool_head(h, batch, params)


# speedup vs baseline: 4.0147x; 3.1438x over previous
import functools

import jax
import jax.numpy as jnp
from jax import lax
from jax.experimental import pallas as pl
from jax.experimental.pallas import tpu as pltpu
from jax.experimental.pallas import tpu_sc as plsc

N = 50000
E = 800000
D_IN = 128
D_E = 16
H = 64
OUT = 64
NUM_GRAPHS = 64
BN_EPS = 1e-5

FC = 32
W = 128
SB = 10
NUM_SC = 2
NUM_TILES = 16
EROWS = E // W
TROWS = 400
NROWS_TILE = N // NUM_TILES


def _edge_e_body(a_ref, w_ref, b_ref, o_ref):
    o_ref[...] = jax.lax.dot(a_ref[...], w_ref[...]) + b_ref[...]


def _edge_e(edge_attr, we, be):
    dout = we.shape[1]
    be_blk = 8000
    return pl.pallas_call(
        _edge_e_body,
        grid=(E // be_blk,),
        in_specs=[
            pl.BlockSpec((be_blk, D_E), lambda i: (i, 0)),
            pl.BlockSpec((D_E, dout), lambda i: (0, 0)),
            pl.BlockSpec((1, dout), lambda i: (0, 0)),
        ],
        out_specs=pl.BlockSpec((be_blk, dout), lambda i: (i, 0)),
        out_shape=jax.ShapeDtypeStruct((E, dout), jnp.float32),
    )(edge_attr, we, be.reshape(1, dout))


def _sc_gather_scatter(h_nat, e_nat, src2d, dst2d, zeros_chunk):
    hin = h_nat.shape[1]
    nc = hin // FC
    per_core = nc // NUM_SC
    mesh = plsc.VectorSubcoreMesh(core_axis_name="c", subcore_axis_name="s")

    @functools.partial(
        pl.kernel,
        out_type=[
            jax.ShapeDtypeStruct((N, hin), jnp.float32),
            jax.ShapeDtypeStruct((nc, N, FC), jnp.float32),
        ],
        mesh=mesh,
        scratch_types=[
            pltpu.VMEM_SHARED((N, FC), jnp.float32),
            pltpu.VMEM((SB, W), jnp.int32),
            pltpu.VMEM((SB, W), jnp.int32),
            pltpu.VMEM((W, FC), jnp.float32),
            pltpu.VMEM((W, FC), jnp.float32),
            pltpu.VMEM((W, FC), jnp.float32),
            pltpu.VMEM((W, FC), jnp.float32),
            pltpu.VMEM((W, FC), jnp.float32),
            pltpu.VMEM((W, FC), jnp.float32),
            pltpu.SemaphoreType.DMA,
            pltpu.SemaphoreType.DMA,
            pltpu.SemaphoreType.DMA,
            pltpu.SemaphoreType.DMA,
            pltpu.SemaphoreType.DMA,
            pltpu.SemaphoreType.DMA,
        ],
        compiler_params=pltpu.CompilerParams(use_tc_tiling_on_sc=False),
    )
    def body(h_hbm, e_hbm, src_hbm, dst_hbm, z_hbm, agg_hbm, ht_hbm,
             acc_sh, sidx, didx, hs0, hs1, ev0, ev1, mb0, mb1,
             sg0, sg1, se0, se1, ss0, ss1):
        core = lax.axis_index("c")
        tid = lax.axis_index("s")
        hs = (hs0, hs1)
        ev = (ev0, ev1)
        mb = (mb0, mb1)
        sg = (sg0, sg1)
        se = (se0, se1)
        ss = (ss0, ss1)

        trow0 = tid * TROWS
        nsb = jnp.where(tid == NUM_TILES - 1, 250 // SB, TROWS // SB)
        r0 = tid * NROWS_TILE
        rsl = pl.ds(r0, NROWS_TILE)

        for k in range(per_core):
            ci = core * per_core + k
            lsl = pl.ds(ci * FC, FC)

            pltpu.sync_copy(h_hbm.at[rsl, lsl], acc_sh.at[rsl])
            pltpu.sync_copy(acc_sh.at[rsl], ht_hbm.at[ci].at[rsl])
            pltpu.sync_copy(z_hbm.at[rsl], acc_sh.at[rsl])
            plsc.subcore_barrier()

            @pl.loop(0, nsb)
            def _sb(s):
                row0 = trow0 + s * SB
                pltpu.sync_copy(src_hbm.at[pl.ds(row0, SB), :], sidx)
                pltpu.sync_copy(dst_hbm.at[pl.ds(row0, SB), :], didx)

                gh = [None, None]
                eh = [None, None]
                sh = [None, None]

                def issue(w):
                    b = w % 2
                    e0 = (row0 + w) * W
                    gh[b] = pltpu.async_copy(
                        ht_hbm.at[ci].at[sidx.at[w]], hs[b], sg[b])
                    eh[b] = pltpu.async_copy(
                        e_hbm.at[pl.ds(e0, W), lsl], ev[b], se[b])

                issue(0)
                issue(1)
                for w in range(SB):
                    b = w % 2
                    gh[b].wait()
                    eh[b].wait()
                    if sh[b] is not None:
                        sh[b].wait()

                    @pl.loop(0, W)
                    def _row(i):
                        for j in range(FC // 16):
                            sl = pl.ds(j * 16, 16)
                            mb[b][i, sl] = jnp.maximum(
                                hs[b][i, sl] + ev[b][i, sl], 0.0)

                    sh[b] = pltpu.async_copy(
                        mb[b], acc_sh.at[didx.at[w]], ss[b], add=True)
                    if w + 2 < SB:
                        issue(w + 2)
                sh[0].wait()
                sh[1].wait()

            plsc.subcore_barrier()
            pltpu.sync_copy(acc_sh.at[rsl], agg_hbm.at[rsl, lsl])
            plsc.subcore_barrier()

    return body(h_nat, e_nat, src2d, dst2d, zeros_chunk)[0]


def _mlp_body(h_ref, a_ref, w1_ref, b1_ref, w2_ref, b2_ref, g_ref, be_ref,
              o_ref):
    z = h_ref[...] + a_ref[...]
    y = jnp.maximum(jax.lax.dot(z, w1_ref[...])
                    + b1_ref[...], 0.0)
    y = jnp.maximum(jax.lax.dot(y, w2_ref[...])
                    + b2_ref[...], 0.0)
    o_ref[...] = y * (1.0 / jnp.sqrt(1.0 + BN_EPS)) * g_ref[...] + be_ref[...]


def _node_mlp(h_nat, agg_nat, p):
    din = h_nat.shape[1]
    bn = 2000
    return pl.pallas_call(
        _mlp_body,
        grid=(N // bn,),
        in_specs=[
            pl.BlockSpec((bn, din), lambda i: (i, 0)),
            pl.BlockSpec((bn, din), lambda i: (i, 0)),
            pl.BlockSpec((din, H), lambda i: (0, 0)),
            pl.BlockSpec((1, H), lambda i: (0, 0)),
            pl.BlockSpec((H, H), lambda i: (0, 0)),
            pl.BlockSpec((1, H), lambda i: (0, 0)),
            pl.BlockSpec((1, H), lambda i: (0, 0)),
            pl.BlockSpec((1, H), lambda i: (0, 0)),
        ],
        out_specs=pl.BlockSpec((bn, H), lambda i: (i, 0)),
        out_shape=jax.ShapeDtypeStruct((N, H), jnp.float32),
    )(h_nat, agg_nat, p["w1"], p["b1"].reshape(1, H), p["w2"],
      p["b2"].reshape(1, H), p["gamma"].reshape(1, H), p["beta"].reshape(1, H))


def _pool_body(h_ref, b_ref, w1_ref, b1_ref, w2_ref, b2_ref, o_ref,
               sums_ref, cnt_ref):
    i = pl.program_id(0)

    @pl.when(i == 0)
    def _init():
        sums_ref[...] = jnp.zeros_like(sums_ref)
        cnt_ref[...] = jnp.zeros_like(cnt_ref)

    ids = b_ref[0]
    onehot = (ids == lax.broadcasted_iota(jnp.int32, (1, NUM_GRAPHS),
                                          1)).astype(jnp.float32)
    sums_ref[...] += jax.lax.dot_general(
        onehot, h_ref[...], (((0,), (0,)), ((), ())))
    ones_col = jnp.ones((onehot.shape[0], 8), jnp.float32)
    cnt_ref[...] += jax.lax.dot_general(
        onehot, ones_col, (((0,), (0,)), ((), ())))

    @pl.when(i == pl.num_programs(0) - 1)
    def _head():
        g = sums_ref[...] / jnp.maximum(cnt_ref[:, 0:1], 1.0)
        g = jnp.maximum(jax.lax.dot(g, w1_ref[...])
                        + b1_ref[...], 0.0)
        o_ref[...] = jax.lax.dot(g, w2_ref[...]) \
            + b2_ref[...]


def _pool_head(h_nat, batch, params):
    bn = 2000
    nb = N // bn
    batch3 = batch.reshape(nb, bn, 1)
    return pl.pallas_call(
        _pool_body,
        grid=(nb,),
        in_specs=[
            pl.BlockSpec((bn, H), lambda i: (i, 0)),
            pl.BlockSpec((1, bn, 1), lambda i: (i, 0, 0)),
            pl.BlockSpec((H, H), lambda i: (0, 0)),
            pl.BlockSpec((1, H), lambda i: (0, 0)),
            pl.BlockSpec((H, OUT), lambda i: (0, 0)),
            pl.BlockSpec((1, OUT), lambda i: (0, 0)),
        ],
        out_specs=pl.BlockSpec((NUM_GRAPHS, OUT), lambda i: (0, 0)),
        out_shape=jax.ShapeDtypeStruct((NUM_GRAPHS, OUT), jnp.float32),
        scratch_shapes=[
            pltpu.VMEM((NUM_GRAPHS, H), jnp.float32),
            pltpu.VMEM((NUM_GRAPHS, 8), jnp.float32),
        ],
    )(h_nat, batch3, params["lin1_w"], params["lin1_b"].reshape(1, H),
      params["lin2_w"], params["lin2_b"].reshape(1, OUT))


def kernel(x, edge_index, edge_attr, batch, params):
    src2d = edge_index[0].reshape(EROWS, W)
    dst2d = edge_index[1].reshape(EROWS, W)
    zeros_chunk = jnp.zeros((N, FC), jnp.float32)

    h = x
    for p in params["layers"]:
        e = _edge_e(edge_attr, p["we"], p["be"])
        agg = _sc_gather_scatter(h, e, src2d, dst2d, zeros_chunk)
        h = _node_mlp(h, agg, p)
    return _pool_head(h, batch, params)


# speedup vs baseline: 4.3508x; 1.0837x over previous
import functools

import jax
import jax.numpy as jnp
from jax import lax
from jax.experimental import pallas as pl
from jax.experimental.pallas import tpu as pltpu
from jax.experimental.pallas import tpu_sc as plsc

N = 50000
E = 800000
D_IN = 128
D_E = 16
H = 64
OUT = 64
NUM_GRAPHS = 64
BN_EPS = 1e-5

FC = 32
W = 128
SB = 10
NUM_SC = 2
NUM_TILES = 16
EROWS = E // W
TROWS = 400
NROWS_TILE = N // NUM_TILES


def _edge_e_body(dout, a_ref, w_ref, b_ref, o_ref):
    y = jax.lax.dot(a_ref[...], w_ref[...]) + b_ref[...]
    if dout < 128:
        y = jnp.concatenate(
            [y, jnp.zeros((y.shape[0], 128 - dout), jnp.float32)], axis=1)
    o_ref[...] = y


def _edge_e(edge_attr, we, be):
    dout = we.shape[1]
    be_blk = 8000
    return pl.pallas_call(
        functools.partial(_edge_e_body, dout),
        grid=(E // be_blk,),
        in_specs=[
            pl.BlockSpec((be_blk, D_E), lambda i: (i, 0)),
            pl.BlockSpec((D_E, dout), lambda i: (0, 0)),
            pl.BlockSpec((1, dout), lambda i: (0, 0)),
        ],
        out_specs=pl.BlockSpec((be_blk, 128), lambda i: (i, 0)),
        out_shape=jax.ShapeDtypeStruct((E, 128), jnp.float32),
    )(edge_attr, we, be.reshape(1, dout))


def _sc_gather_scatter(h_nat, e_nat, src2d, dst2d, zeros_chunk, hin):
    nc = hin // FC
    per_core = nc // NUM_SC
    mesh = plsc.VectorSubcoreMesh(core_axis_name="c", subcore_axis_name="s")

    @functools.partial(
        pl.kernel,
        out_type=[
            jax.ShapeDtypeStruct((N, 128), jnp.float32),
            jax.ShapeDtypeStruct((nc, N, FC), jnp.float32),
        ],
        mesh=mesh,
        scratch_types=[
            pltpu.VMEM_SHARED((N, FC), jnp.float32),
            pltpu.VMEM((SB, W), jnp.int32),
            pltpu.VMEM((SB, W), jnp.int32),
            pltpu.VMEM((W, FC), jnp.float32),
            pltpu.VMEM((W, FC), jnp.float32),
            pltpu.VMEM((W, FC), jnp.float32),
            pltpu.VMEM((W, FC), jnp.float32),
            pltpu.VMEM((W, FC), jnp.float32),
            pltpu.VMEM((W, FC), jnp.float32),
            pltpu.SemaphoreType.DMA,
            pltpu.SemaphoreType.DMA,
            pltpu.SemaphoreType.DMA,
            pltpu.SemaphoreType.DMA,
            pltpu.SemaphoreType.DMA,
            pltpu.SemaphoreType.DMA,
        ],
        compiler_params=pltpu.CompilerParams(use_tc_tiling_on_sc=False),
    )
    def body(h_hbm, e_hbm, src_hbm, dst_hbm, z_hbm, agg_hbm, ht_hbm,
             acc_sh, sidx, didx, hs0, hs1, ev0, ev1, mb0, mb1,
             sg0, sg1, se0, se1, ss0, ss1):
        core = lax.axis_index("c")
        tid = lax.axis_index("s")
        hs = (hs0, hs1)
        ev = (ev0, ev1)
        mb = (mb0, mb1)
        sg = (sg0, sg1)
        se = (se0, se1)
        ss = (ss0, ss1)

        trow0 = tid * TROWS
        nsb = jnp.where(tid == NUM_TILES - 1, 250 // SB, TROWS // SB)
        r0 = tid * NROWS_TILE
        rsl = pl.ds(r0, NROWS_TILE)

        for k in range(per_core):
            ci = core * per_core + k
            lsl = pl.ds(ci * FC, FC)

            pltpu.sync_copy(h_hbm.at[rsl, lsl], acc_sh.at[rsl])
            pltpu.sync_copy(acc_sh.at[rsl], ht_hbm.at[ci].at[rsl])
            pltpu.sync_copy(z_hbm.at[rsl], acc_sh.at[rsl])
            plsc.subcore_barrier()

            @pl.loop(0, nsb)
            def _sb(s):
                row0 = trow0 + s * SB
                pltpu.sync_copy(src_hbm.at[pl.ds(row0, SB), :], sidx)
                pltpu.sync_copy(dst_hbm.at[pl.ds(row0, SB), :], didx)

                gh = [None, None]
                eh = [None, None]
                sh = [None, None]

                def issue(w):
                    b = w % 2
                    e0 = (row0 + w) * W
                    gh[b] = pltpu.async_copy(
                        ht_hbm.at[ci].at[sidx.at[w]], hs[b], sg[b])
                    eh[b] = pltpu.async_copy(
                        e_hbm.at[pl.ds(e0, W), lsl], ev[b], se[b])

                issue(0)
                issue(1)
                for w in range(SB):
                    b = w % 2
                    gh[b].wait()
                    eh[b].wait()
                    if sh[b] is not None:
                        sh[b].wait()

                    @pl.loop(0, W)
                    def _row(i):
                        for j in range(FC // 16):
                            sl = pl.ds(j * 16, 16)
                            mb[b][i, sl] = jnp.maximum(
                                hs[b][i, sl] + ev[b][i, sl], 0.0)

                    sh[b] = pltpu.async_copy(
                        mb[b], acc_sh.at[didx.at[w]], ss[b], add=True)
                    if w + 2 < SB:
                        issue(w + 2)
                sh[0].wait()
                sh[1].wait()

            plsc.subcore_barrier()
            pltpu.sync_copy(acc_sh.at[rsl], agg_hbm.at[rsl, lsl])
            plsc.subcore_barrier()

    return body(h_nat, e_nat, src2d, dst2d, zeros_chunk)[0]


def _mlp_body(din, h_ref, a_ref, w1_ref, b1_ref, w2_ref, b2_ref, g_ref,
              be_ref, o_ref):
    z = h_ref[:, :din] + a_ref[:, :din]
    y = jnp.maximum(jax.lax.dot(z, w1_ref[...])
                    + b1_ref[...], 0.0)
    y = jnp.maximum(jax.lax.dot(y, w2_ref[...])
                    + b2_ref[...], 0.0)
    y = y * (1.0 / jnp.sqrt(1.0 + BN_EPS)) * g_ref[...] + be_ref[...]
    o_ref[...] = jnp.concatenate(
        [y, jnp.zeros((y.shape[0], 128 - H), jnp.float32)], axis=1)


def _node_mlp(h_nat, agg_nat, p):
    din = p["w1"].shape[0]
    bn = 2000
    return pl.pallas_call(
        functools.partial(_mlp_body, din),
        grid=(N // bn,),
        in_specs=[
            pl.BlockSpec((bn, 128), lambda i: (i, 0)),
            pl.BlockSpec((bn, 128), lambda i: (i, 0)),
            pl.BlockSpec((din, H), lambda i: (0, 0)),
            pl.BlockSpec((1, H), lambda i: (0, 0)),
            pl.BlockSpec((H, H), lambda i: (0, 0)),
            pl.BlockSpec((1, H), lambda i: (0, 0)),
            pl.BlockSpec((1, H), lambda i: (0, 0)),
            pl.BlockSpec((1, H), lambda i: (0, 0)),
        ],
        out_specs=pl.BlockSpec((bn, 128), lambda i: (i, 0)),
        out_shape=jax.ShapeDtypeStruct((N, 128), jnp.float32),
    )(h_nat, agg_nat, p["w1"], p["b1"].reshape(1, H), p["w2"],
      p["b2"].reshape(1, H), p["gamma"].reshape(1, H), p["beta"].reshape(1, H))


def _pool_body(h_ref, b_ref, w1_ref, b1_ref, w2_ref, b2_ref, o_ref,
               sums_ref, cnt_ref):
    i = pl.program_id(0)

    @pl.when(i == 0)
    def _init():
        sums_ref[...] = jnp.zeros_like(sums_ref)
        cnt_ref[...] = jnp.zeros_like(cnt_ref)

    ids = b_ref[0]
    onehot = (ids == lax.broadcasted_iota(jnp.int32, (1, NUM_GRAPHS),
                                          1)).astype(jnp.float32)
    sums_ref[...] += jax.lax.dot_general(
        onehot, h_ref[:, :H], (((0,), (0,)), ((), ())))
    ones_col = jnp.ones((onehot.shape[0], 8), jnp.float32)
    cnt_ref[...] += jax.lax.dot_general(
        onehot, ones_col, (((0,), (0,)), ((), ())))

    @pl.when(i == pl.num_programs(0) - 1)
    def _head():
        g = sums_ref[...] / jnp.maximum(cnt_ref[:, 0:1], 1.0)
        g = jnp.maximum(jax.lax.dot(g, w1_ref[...])
                        + b1_ref[...], 0.0)
        o_ref[...] = jax.lax.dot(g, w2_ref[...]) \
            + b2_ref[...]


def _pool_head(h_nat, batch, params):
    bn = 2000
    nb = N // bn
    batch3 = batch.reshape(nb, bn, 1)
    return pl.pallas_call(
        _pool_body,
        grid=(nb,),
        in_specs=[
            pl.BlockSpec((bn, 128), lambda i: (i, 0)),
            pl.BlockSpec((1, bn, 1), lambda i: (i, 0, 0)),
            pl.BlockSpec((H, H), lambda i: (0, 0)),
            pl.BlockSpec((1, H), lambda i: (0, 0)),
            pl.BlockSpec((H, OUT), lambda i: (0, 0)),
            pl.BlockSpec((1, OUT), lambda i: (0, 0)),
        ],
        out_specs=pl.BlockSpec((NUM_GRAPHS, OUT), lambda i: (0, 0)),
        out_shape=jax.ShapeDtypeStruct((NUM_GRAPHS, OUT), jnp.float32),
        scratch_shapes=[
            pltpu.VMEM((NUM_GRAPHS, H), jnp.float32),
            pltpu.VMEM((NUM_GRAPHS, 8), jnp.float32),
        ],
    )(h_nat, batch3, params["lin1_w"], params["lin1_b"].reshape(1, H),
      params["lin2_w"], params["lin2_b"].reshape(1, OUT))


def kernel(x, edge_index, edge_attr, batch, params):
    src2d = edge_index[0].reshape(EROWS, W)
    dst2d = edge_index[1].reshape(EROWS, W)
    zeros_chunk = jnp.zeros((N, FC), jnp.float32)

    h = x
    for p in params["layers"]:
        e = _edge_e(edge_attr, p["we"], p["be"])
        agg = _sc_gather_scatter(h, e, src2d, dst2d, zeros_chunk,
                                 p["w1"].shape[0])
        h = _node_mlp(h, agg, p)
    return _pool_head(h, batch, params)


# speedup vs baseline: 4.6530x; 1.0695x over previous
import functools

import jax
import jax.numpy as jnp
from jax import lax
from jax.experimental import pallas as pl
from jax.experimental.pallas import tpu as pltpu
from jax.experimental.pallas import tpu_sc as plsc

N = 50000
E = 800000
D_IN = 128
D_E = 16
H = 64
OUT = 64
NUM_GRAPHS = 64
BN_EPS = 1e-5

FC = 32
W = 128
SB = 25
NUM_SC = 2
NUM_TILES = 16
EROWS = E // W
TROWS = 400
NROWS_TILE = N // NUM_TILES


def _edge_e_body(dout, a_ref, w_ref, b_ref, o_ref):
    y = jax.lax.dot(a_ref[...], w_ref[...]) + b_ref[...]
    if dout < 128:
        y = jnp.concatenate(
            [y, jnp.zeros((y.shape[0], 128 - dout), jnp.float32)], axis=1)
    o_ref[...] = y


def _edge_e(edge_attr, we, be):
    dout = we.shape[1]
    be_blk = 8000
    return pl.pallas_call(
        functools.partial(_edge_e_body, dout),
        grid=(E // be_blk,),
        in_specs=[
            pl.BlockSpec((be_blk, D_E), lambda i: (i, 0)),
            pl.BlockSpec((D_E, dout), lambda i: (0, 0)),
            pl.BlockSpec((1, dout), lambda i: (0, 0)),
        ],
        out_specs=pl.BlockSpec((be_blk, 128), lambda i: (i, 0)),
        out_shape=jax.ShapeDtypeStruct((E, 128), jnp.float32),
    )(edge_attr, we, be.reshape(1, dout))


def _sc_gather_scatter(h_nat, e_nat, src2d, dst2d, zeros_chunk, hin):
    nc = hin // FC
    per_core = nc // NUM_SC
    mesh = plsc.VectorSubcoreMesh(core_axis_name="c", subcore_axis_name="s")

    @functools.partial(
        pl.kernel,
        out_type=[
            jax.ShapeDtypeStruct((N, 128), jnp.float32),
            jax.ShapeDtypeStruct((nc, N, FC), jnp.float32),
        ],
        mesh=mesh,
        scratch_types=[
            pltpu.VMEM_SHARED((N, FC), jnp.float32),
            pltpu.VMEM((SB, W), jnp.int32),
            pltpu.VMEM((SB, W), jnp.int32),
            pltpu.VMEM((W, FC), jnp.float32),
            pltpu.VMEM((W, FC), jnp.float32),
            pltpu.VMEM((W, FC), jnp.float32),
            pltpu.VMEM((W, FC), jnp.float32),
            pltpu.VMEM((W, FC), jnp.float32),
            pltpu.VMEM((W, FC), jnp.float32),
            pltpu.SemaphoreType.DMA,
            pltpu.SemaphoreType.DMA,
            pltpu.SemaphoreType.DMA,
            pltpu.SemaphoreType.DMA,
            pltpu.SemaphoreType.DMA,
            pltpu.SemaphoreType.DMA,
        ],
        compiler_params=pltpu.CompilerParams(use_tc_tiling_on_sc=False),
    )
    def body(h_hbm, e_hbm, src_hbm, dst_hbm, z_hbm, agg_hbm, ht_hbm,
             acc_sh, sidx, didx, hs0, hs1, ev0, ev1, mb0, mb1,
             sg0, sg1, se0, se1, ss0, ss1):
        core = lax.axis_index("c")
        tid = lax.axis_index("s")
        hs = (hs0, hs1)
        ev = (ev0, ev1)
        mb = (mb0, mb1)
        sg = (sg0, sg1)
        se = (se0, se1)
        ss = (ss0, ss1)

        trow0 = tid * TROWS
        nsb = jnp.where(tid == NUM_TILES - 1, 250 // SB, TROWS // SB)
        r0 = tid * NROWS_TILE
        rsl = pl.ds(r0, NROWS_TILE)

        for k in range(per_core):
            ci = core * per_core + k
            lsl = pl.ds(ci * FC, FC)

            pltpu.sync_copy(h_hbm.at[rsl, lsl], acc_sh.at[rsl])
            pltpu.sync_copy(acc_sh.at[rsl], ht_hbm.at[ci].at[rsl])
            pltpu.sync_copy(z_hbm.at[rsl], acc_sh.at[rsl])
            plsc.subcore_barrier()

            @pl.loop(0, nsb)
            def _sb(s):
                row0 = trow0 + s * SB
                pltpu.sync_copy(src_hbm.at[pl.ds(row0, SB), :], sidx)
                pltpu.sync_copy(dst_hbm.at[pl.ds(row0, SB), :], didx)

                gh = [None, None]
                eh = [None, None]
                sh = [None, None]

                def issue(w):
                    b = w % 2
                    e0 = (row0 + w) * W
                    gh[b] = pltpu.async_copy(
                        ht_hbm.at[ci].at[sidx.at[w]], hs[b], sg[b])
                    eh[b] = pltpu.async_copy(
                        e_hbm.at[pl.ds(e0, W), lsl], ev[b], se[b])

                issue(0)
                issue(1)
                for w in range(SB):
                    b = w % 2
                    gh[b].wait()
                    eh[b].wait()
                    if sh[b] is not None:
                        sh[b].wait()

                    @pl.loop(0, W)
                    def _row(i):
                        for j in range(FC // 16):
                            sl = pl.ds(j * 16, 16)
                            mb[b][i, sl] = jnp.maximum(
                                hs[b][i, sl] + ev[b][i, sl], 0.0)

                    sh[b] = pltpu.async_copy(
                        mb[b], acc_sh.at[didx.at[w]], ss[b], add=True)
                    if w + 2 < SB:
                        issue(w + 2)
                sh[0].wait()
                sh[1].wait()

            plsc.subcore_barrier()
            pltpu.sync_copy(acc_sh.at[rsl], agg_hbm.at[rsl, lsl])
            plsc.subcore_barrier()

    return body(h_nat, e_nat, src2d, dst2d, zeros_chunk)[0]


def _mlp_body(din, h_ref, a_ref, w1_ref, b1_ref, w2_ref, b2_ref, g_ref,
              be_ref, o_ref):
    z = h_ref[:, :din] + a_ref[:, :din]
    y = jnp.maximum(jax.lax.dot(z, w1_ref[...])
                    + b1_ref[...], 0.0)
    y = jnp.maximum(jax.lax.dot(y, w2_ref[...])
                    + b2_ref[...], 0.0)
    y = y * (1.0 / jnp.sqrt(1.0 + BN_EPS)) * g_ref[...] + be_ref[...]
    o_ref[...] = jnp.concatenate(
        [y, jnp.zeros((y.shape[0], 128 - H), jnp.float32)], axis=1)


def _node_mlp(h_nat, agg_nat, p):
    din = p["w1"].shape[0]
    bn = 2000
    return pl.pallas_call(
        functools.partial(_mlp_body, din),
        grid=(N // bn,),
        in_specs=[
            pl.BlockSpec((bn, 128), lambda i: (i, 0)),
            pl.BlockSpec((bn, 128), lambda i: (i, 0)),
            pl.BlockSpec((din, H), lambda i: (0, 0)),
            pl.BlockSpec((1, H), lambda i: (0, 0)),
            pl.BlockSpec((H, H), lambda i: (0, 0)),
            pl.BlockSpec((1, H), lambda i: (0, 0)),
            pl.BlockSpec((1, H), lambda i: (0, 0)),
            pl.BlockSpec((1, H), lambda i: (0, 0)),
        ],
        out_specs=pl.BlockSpec((bn, 128), lambda i: (i, 0)),
        out_shape=jax.ShapeDtypeStruct((N, 128), jnp.float32),
    )(h_nat, agg_nat, p["w1"], p["b1"].reshape(1, H), p["w2"],
      p["b2"].reshape(1, H), p["gamma"].reshape(1, H), p["beta"].reshape(1, H))


def _pool_body(h_ref, b_ref, w1_ref, b1_ref, w2_ref, b2_ref, o_ref,
               sums_ref, cnt_ref):
    i = pl.program_id(0)

    @pl.when(i == 0)
    def _init():
        sums_ref[...] = jnp.zeros_like(sums_ref)
        cnt_ref[...] = jnp.zeros_like(cnt_ref)

    ids = b_ref[0]
    onehot = (ids == lax.broadcasted_iota(jnp.int32, (1, NUM_GRAPHS),
                                          1)).astype(jnp.float32)
    sums_ref[...] += jax.lax.dot_general(
        onehot, h_ref[:, :H], (((0,), (0,)), ((), ())))
    ones_col = jnp.ones((onehot.shape[0], 8), jnp.float32)
    cnt_ref[...] += jax.lax.dot_general(
        onehot, ones_col, (((0,), (0,)), ((), ())))

    @pl.when(i == pl.num_programs(0) - 1)
    def _head():
        g = sums_ref[...] / jnp.maximum(cnt_ref[:, 0:1], 1.0)
        g = jnp.maximum(jax.lax.dot(g, w1_ref[...])
                        + b1_ref[...], 0.0)
        o_ref[...] = jax.lax.dot(g, w2_ref[...]) \
            + b2_ref[...]


def _pool_head(h_nat, batch, params):
    bn = 2000
    nb = N // bn
    batch3 = batch.reshape(nb, bn, 1)
    return pl.pallas_call(
        _pool_body,
        grid=(nb,),
        in_specs=[
            pl.BlockSpec((bn, 128), lambda i: (i, 0)),
            pl.BlockSpec((1, bn, 1), lambda i: (i, 0, 0)),
            pl.BlockSpec((H, H), lambda i: (0, 0)),
            pl.BlockSpec((1, H), lambda i: (0, 0)),
            pl.BlockSpec((H, OUT), lambda i: (0, 0)),
            pl.BlockSpec((1, OUT), lambda i: (0, 0)),
        ],
        out_specs=pl.BlockSpec((NUM_GRAPHS, OUT), lambda i: (0, 0)),
        out_shape=jax.ShapeDtypeStruct((NUM_GRAPHS, OUT), jnp.float32),
        scratch_shapes=[
            pltpu.VMEM((NUM_GRAPHS, H), jnp.float32),
            pltpu.VMEM((NUM_GRAPHS, 8), jnp.float32),
        ],
    )(h_nat, batch3, params["lin1_w"], params["lin1_b"].reshape(1, H),
      params["lin2_w"], params["lin2_b"].reshape(1, OUT))


def kernel(x, edge_index, edge_attr, batch, params):
    src2d = edge_index[0].reshape(EROWS, W)
    dst2d = edge_index[1].reshape(EROWS, W)
    zeros_chunk = jnp.zeros((N, FC), jnp.float32)

    h = x
    for p in params["layers"]:
        e = _edge_e(edge_attr, p["we"], p["be"])
        agg = _sc_gather_scatter(h, e, src2d, dst2d, zeros_chunk,
                                 p["w1"].shape[0])
        h = _node_mlp(h, agg, p)
    return _pool_head(h, batch, params)
